# Initial kernel scaffold; baseline (speedup 1.0000x reference)
import jax
import jax.numpy as jnp
from jax.experimental import pallas as pl


def kernel(x, neigh_x, kernel, bias):
    raise NotImplementedError("write your pallas kernel here")


# speedup vs baseline: 10.4356x; 10.4356x over previous
import jax
import jax.numpy as jnp
from jax.experimental import pallas as pl


def _oem_pairs(n):
    pairs = []
    p = 1
    while p < n:
        k = p
        while k >= 1:
            for j in range(k % p, n - k, 2 * k):
                for i in range(0, min(k, n - j - k)):
                    if (i + j) // (2 * p) == (i + j + k) // (2 * p):
                        pairs.append((i + j, i + j + k))
            k //= 2
        p *= 2
    return pairs


_PAIRS16 = _oem_pairs(16)


def _tree_reduce(vals, op):
    vals = list(vals)
    while len(vals) > 1:
        nxt = [op(vals[i], vals[i + 1]) for i in range(0, len(vals) - 1, 2)]
        if len(vals) % 2:
            nxt.append(vals[-1])
        vals = nxt
    return vals[0]


def _median33(vals):
    a = list(vals[:16])
    b = list(vals[16:32])
    e = vals[32]
    for i, j in _PAIRS16:
        lo = jnp.minimum(a[i], a[j])
        a[j] = jnp.maximum(a[i], a[j])
        a[i] = lo
        lo = jnp.minimum(b[i], b[j])
        b[j] = jnp.maximum(b[i], b[j])
        b[i] = lo
    lo = [jnp.minimum(a[i], b[15 - i]) for i in range(16)]
    hi = [jnp.maximum(a[i], b[15 - i]) for i in range(16)]
    v15 = _tree_reduce(lo, jnp.maximum)
    v16 = _tree_reduce(hi, jnp.minimum)
    return jnp.minimum(jnp.maximum(e, v15), v16)


def _body(x_ref, nb_ref, w_ref, b_ref, o_ref, *, bn, s):
    meds = []
    for c in range(bn // 8):
        rows = pl.ds(c * 8, 8)
        vals = [nb_ref[rows, j, :] for j in range(s)]
        vals.append(x_ref[rows, :])
        meds.append(_median33(vals))
    med = jnp.concatenate(meds, axis=0)
    o_ref[...] = (
        jnp.dot(med, w_ref[...], preferred_element_type=jnp.float32) + b_ref[...]
    )


def kernel(x, neigh_x, kernel, bias):
    n, s, d = neigh_x.shape
    units = kernel.shape[1]
    assert s == 32, "median network is specialized to 32 neighbors + self"
    bn = 80
    assert n % bn == 0
    bias2 = bias.reshape(1, units)

    import functools

    body = functools.partial(_body, bn=bn, s=s)
    return pl.pallas_call(
        body,
        grid=(n // bn,),
        in_specs=[
            pl.BlockSpec((bn, d), lambda i: (i, 0)),
            pl.BlockSpec((bn, s, d), lambda i: (i, 0, 0)),
            pl.BlockSpec((d, units), lambda i: (0, 0)),
            pl.BlockSpec((1, units), lambda i: (0, 0)),
        ],
        out_specs=pl.BlockSpec((bn, units), lambda i: (i, 0)),
        out_shape=jax.ShapeDtypeStruct((n, units), jnp.float32),
    )(x, neigh_x, kernel, bias2)


# speedup vs baseline: 48.2341x; 4.6221x over previous
import jax
import jax.numpy as jnp
from jax.experimental import pallas as pl


def _oem_pairs(n):
    pairs = []
    p = 1
    while p < n:
        k = p
        while k >= 1:
            for j in range(k % p, n - k, 2 * k):
                for i in range(0, min(k, n - j - k)):
                    if (i + j) // (2 * p) == (i + j + k) // (2 * p):
                        pairs.append((i + j, i + j + k))
            k //= 2
        p *= 2
    return pairs


_PAIRS16 = _oem_pairs(16)


def _tree_reduce(vals, op):
    vals = list(vals)
    while len(vals) > 1:
        nxt = [op(vals[i], vals[i + 1]) for i in range(0, len(vals) - 1, 2)]
        if len(vals) % 2:
            nxt.append(vals[-1])
        vals = nxt
    return vals[0]


def _median33(vals):
    a = list(vals[:16])
    b = list(vals[16:32])
    e = vals[32]
    for i, j in _PAIRS16:
        lo = jnp.minimum(a[i], a[j])
        a[j] = jnp.maximum(a[i], a[j])
        a[i] = lo
        lo = jnp.minimum(b[i], b[j])
        b[j] = jnp.maximum(b[i], b[j])
        b[i] = lo
    lo = [jnp.minimum(a[i], b[15 - i]) for i in range(16)]
    hi = [jnp.maximum(a[i], b[15 - i]) for i in range(16)]
    v15 = _tree_reduce(lo, jnp.maximum)
    v16 = _tree_reduce(hi, jnp.minimum)
    return jnp.minimum(jnp.maximum(e, v15), v16)


def _body(x_ref, nb_ref, w_ref, b_ref, o_ref, *, bn, s):
    meds = []
    for c in range(bn // 8):
        vals = [nb_ref[pl.Slice(c * 8 * s + j, 8, s), :] for j in range(s)]
        vals.append(x_ref[pl.ds(c * 8, 8), :])
        meds.append(_median33(vals))
    med = jnp.concatenate(meds, axis=0)
    o_ref[...] = (
        jnp.dot(med, w_ref[...], preferred_element_type=jnp.float32) + b_ref[...]
    )


def kernel(x, neigh_x, kernel, bias):
    n, s, d = neigh_x.shape
    units = kernel.shape[1]
    assert s == 32, "median network is specialized to 32 neighbors + self"
    bn = 80
    assert n % bn == 0
    bias2 = bias.reshape(1, units)
    nb_flat = neigh_x.reshape(n * s, d)

    import functools

    body = functools.partial(_body, bn=bn, s=s)
    return pl.pallas_call(
        body,
        grid=(n // bn,),
        in_specs=[
            pl.BlockSpec((bn, d), lambda i: (i, 0)),
            pl.BlockSpec((bn * s, d), lambda i: (i, 0)),
            pl.BlockSpec((d, units), lambda i: (0, 0)),
            pl.BlockSpec((1, units), lambda i: (0, 0)),
        ],
        out_specs=pl.BlockSpec((bn, units), lambda i: (i, 0)),
        out_shape=jax.ShapeDtypeStruct((n, units), jnp.float32),
    )(x, nb_flat, kernel, bias2)


# speedup vs baseline: 55.3048x; 1.1466x over previous
import jax
import jax.numpy as jnp
from jax.experimental import pallas as pl
from jax.experimental.pallas import tpu as pltpu


def _oem_pairs(n):
    pairs = []
    p = 1
    while p < n:
        k = p
        while k >= 1:
            for j in range(k % p, n - k, 2 * k):
                for i in range(0, min(k, n - j - k)):
                    if (i + j) // (2 * p) == (i + j + k) // (2 * p):
                        pairs.append((i + j, i + j + k))
            k //= 2
        p *= 2
    return pairs


_PAIRS16 = _oem_pairs(16)


def _tree_reduce(vals, op):
    vals = list(vals)
    while len(vals) > 1:
        nxt = [op(vals[i], vals[i + 1]) for i in range(0, len(vals) - 1, 2)]
        if len(vals) % 2:
            nxt.append(vals[-1])
        vals = nxt
    return vals[0]


def _median33(vals):
    a = list(vals[:16])
    b = list(vals[16:32])
    e = vals[32]
    for i, j in _PAIRS16:
        lo = jnp.minimum(a[i], a[j])
        a[j] = jnp.maximum(a[i], a[j])
        a[i] = lo
        lo = jnp.minimum(b[i], b[j])
        b[j] = jnp.maximum(b[i], b[j])
        b[i] = lo
    lo = [jnp.minimum(a[i], b[15 - i]) for i in range(16)]
    hi = [jnp.maximum(a[i], b[15 - i]) for i in range(16)]
    v15 = _tree_reduce(lo, jnp.maximum)
    v16 = _tree_reduce(hi, jnp.minimum)
    return jnp.minimum(jnp.maximum(e, v15), v16)


def _sub_iota(d):
    return jax.lax.broadcasted_iota(jnp.int32, (8, d), 0)


def _transpose8(a, d):
    sub = _sub_iota(d)
    v = list(a)
    for k in (4, 2, 1):
        mask = (sub & k) == 0
        nxt = list(v)
        for i in range(8):
            if i & k:
                continue
            j = i + k
            nxt[i] = jnp.where(mask, v[i], pltpu.roll(v[j], k, 0))
            nxt[j] = jnp.where(mask, pltpu.roll(v[i], -k % 8, 0), v[j])
        v = nxt
    return v


def _body(x_ref, nb_ref, w_ref, b_ref, o_ref, *, bn, s):
    d = x_ref.shape[1]
    meds = []
    for c in range(bn // 8):
        tiles = [nb_ref[pl.ds(c * 8 * s + 8 * m, 8), :] for m in range(s)]
        vals = []
        for t in range(4):
            vals.extend(_transpose8([tiles[4 * r + t] for r in range(8)], d))
        vals.append(x_ref[pl.ds(c * 8, 8), :])
        meds.append(_median33(vals))
    med = jnp.concatenate(meds, axis=0)
    o_ref[...] = (
        jnp.dot(med, w_ref[...], preferred_element_type=jnp.float32) + b_ref[...]
    )


def kernel(x, neigh_x, kernel, bias):
    n, s, d = neigh_x.shape
    units = kernel.shape[1]
    assert s == 32, "median network is specialized to 32 neighbors + self"
    bn = 80
    assert n % bn == 0
    bias2 = bias.reshape(1, units)
    nb_flat = neigh_x.reshape(n * s, d)

    import functools

    body = functools.partial(_body, bn=bn, s=s)
    return pl.pallas_call(
        body,
        grid=(n // bn,),
        in_specs=[
            pl.BlockSpec((bn, d), lambda i: (i, 0)),
            pl.BlockSpec((bn * s, d), lambda i: (i, 0)),
            pl.BlockSpec((d, units), lambda i: (0, 0)),
            pl.BlockSpec((1, units), lambda i: (0, 0)),
        ],
        out_specs=pl.BlockSpec((bn, units), lambda i: (i, 0)),
        out_shape=jax.ShapeDtypeStruct((n, units), jnp.float32),
    )(x, nb_flat, kernel, bias2)


# speedup vs baseline: 76.6114x; 1.3853x over previous
import jax
import jax.numpy as jnp
from jax.experimental import pallas as pl
from jax.experimental.pallas import tpu as pltpu


def _oem_pairs(n):
    pairs = []
    p = 1
    while p < n:
        k = p
        while k >= 1:
            for j in range(k % p, n - k, 2 * k):
                for i in range(0, min(k, n - j - k)):
                    if (i + j) // (2 * p) == (i + j + k) // (2 * p):
                        pairs.append((i + j, i + j + k))
            k //= 2
        p *= 2
    return pairs


_PAIRS16 = _oem_pairs(16)


def _tree_reduce(vals, op):
    vals = list(vals)
    while len(vals) > 1:
        nxt = [op(vals[i], vals[i + 1]) for i in range(0, len(vals) - 1, 2)]
        if len(vals) % 2:
            nxt.append(vals[-1])
        vals = nxt
    return vals[0]


def _median33(vals):
    a = list(vals[:16])
    b = list(vals[16:32])
    e = vals[32]
    for i, j in _PAIRS16:
        lo = jnp.minimum(a[i], a[j])
        a[j] = jnp.maximum(a[i], a[j])
        a[i] = lo
        lo = jnp.minimum(b[i], b[j])
        b[j] = jnp.maximum(b[i], b[j])
        b[i] = lo
    lo = [jnp.minimum(a[i], b[15 - i]) for i in range(16)]
    hi = [jnp.maximum(a[i], b[15 - i]) for i in range(16)]
    v15 = _tree_reduce(lo, jnp.maximum)
    v16 = _tree_reduce(hi, jnp.minimum)
    return jnp.minimum(jnp.maximum(e, v15), v16)


def _sub_iota(d):
    return jax.lax.broadcasted_iota(jnp.int32, (8, d), 0)


def _transpose8(a, d):
    sub = _sub_iota(d)
    v = list(a)
    for k in (4, 2, 1):
        mask = (sub & k) == 0
        nxt = list(v)
        for i in range(8):
            if i & k:
                continue
            j = i + k
            nxt[i] = jnp.where(mask, v[i], pltpu.roll(v[j], k, 0))
            nxt[j] = jnp.where(mask, pltpu.roll(v[i], -k % 8, 0), v[j])
        v = nxt
    return v


def _body(x_ref, nb_ref, w_ref, b_ref, o_ref, *, bn, s):
    d = x_ref.shape[1]
    meds = []
    for c in range(bn // 8):
        tiles = [nb_ref[pl.ds(c * 8 * s + 8 * m, 8), :] for m in range(s)]
        vals = []
        for t in range(4):
            vals.extend(_transpose8([tiles[4 * r + t] for r in range(8)], d))
        vals.append(x_ref[pl.ds(c * 8, 8), :])
        meds.append(_median33(vals))
    med = jnp.concatenate(meds, axis=0)
    o_ref[...] = (
        jnp.dot(med, w_ref[...], preferred_element_type=jnp.float32) + b_ref[...]
    )


def kernel(x, neigh_x, kernel, bias):
    n, s, d = neigh_x.shape
    units = kernel.shape[1]
    assert s == 32, "median network is specialized to 32 neighbors + self"
    bn = 200
    assert n % bn == 0
    bias2 = bias.reshape(1, units)
    nb_flat = neigh_x.reshape(n * s, d)

    import functools

    body = functools.partial(_body, bn=bn, s=s)
    return pl.pallas_call(
        body,
        grid=(n // bn,),
        in_specs=[
            pl.BlockSpec((bn, d), lambda i: (i, 0)),
            pl.BlockSpec((bn * s, d), lambda i: (i, 0)),
            pl.BlockSpec((d, units), lambda i: (0, 0)),
            pl.BlockSpec((1, units), lambda i: (0, 0)),
        ],
        out_specs=pl.BlockSpec((bn, units), lambda i: (i, 0)),
        out_shape=jax.ShapeDtypeStruct((n, units), jnp.float32),
    )(x, nb_flat, kernel, bias2)


# speedup vs baseline: 83.7333x; 1.0930x over previous
import jax
import jax.numpy as jnp
from jax.experimental import pallas as pl
from jax.experimental.pallas import tpu as pltpu


def _oem_pairs(n):
    pairs = []
    p = 1
    while p < n:
        k = p
        while k >= 1:
            for j in range(k % p, n - k, 2 * k):
                for i in range(0, min(k, n - j - k)):
                    if (i + j) // (2 * p) == (i + j + k) // (2 * p):
                        pairs.append((i + j, i + j + k))
            k //= 2
        p *= 2
    return pairs


_PAIRS16 = _oem_pairs(16)


def _tree_reduce(vals, op):
    vals = list(vals)
    while len(vals) > 1:
        nxt = [op(vals[i], vals[i + 1]) for i in range(0, len(vals) - 1, 2)]
        if len(vals) % 2:
            nxt.append(vals[-1])
        vals = nxt
    return vals[0]


def _median33(vals):
    a = list(vals[:16])
    b = list(vals[16:32])
    e = vals[32]
    for i, j in _PAIRS16:
        lo = jnp.minimum(a[i], a[j])
        a[j] = jnp.maximum(a[i], a[j])
        a[i] = lo
        lo = jnp.minimum(b[i], b[j])
        b[j] = jnp.maximum(b[i], b[j])
        b[i] = lo
    lo = [jnp.minimum(a[i], b[15 - i]) for i in range(16)]
    hi = [jnp.maximum(a[i], b[15 - i]) for i in range(16)]
    v15 = _tree_reduce(lo, jnp.maximum)
    v16 = _tree_reduce(hi, jnp.minimum)
    return jnp.minimum(jnp.maximum(e, v15), v16)


def _sub_iota(d):
    return jax.lax.broadcasted_iota(jnp.int32, (8, d), 0)


def _transpose8(a, d):
    sub = _sub_iota(d)
    v = list(a)
    for k in (4, 2, 1):
        mask = (sub & k) == 0
        nxt = list(v)
        for i in range(8):
            if i & k:
                continue
            j = i + k
            nxt[i] = jnp.where(mask, v[i], pltpu.roll(v[j], k, 0))
            nxt[j] = jnp.where(mask, pltpu.roll(v[i], -k % 8, 0), v[j])
        v = nxt
    return v


def _body(x_ref, nb_ref, w_ref, b_ref, o_ref, *, bn, s):
    d = x_ref.shape[1]
    meds = []
    for c in range(bn // 8):
        tiles = [nb_ref[pl.ds(c * 8 * s + 8 * m, 8), :] for m in range(s)]
        vals = []
        for t in range(4):
            vals.extend(_transpose8([tiles[4 * r + t] for r in range(8)], d))
        vals.append(x_ref[pl.ds(c * 8, 8), :])
        meds.append(_median33(vals))
    med = jnp.concatenate(meds, axis=0)
    o_ref[...] = (
        jnp.dot(med, w_ref[...], preferred_element_type=jnp.float32) + b_ref[...]
    )


def kernel(x, neigh_x, kernel, bias):
    n, s, d = neigh_x.shape
    units = kernel.shape[1]
    assert s == 32, "median network is specialized to 32 neighbors + self"
    bn = 400
    assert n % bn == 0
    bias2 = bias.reshape(1, units)
    nb_flat = neigh_x.reshape(n * s, d)

    import functools

    body = functools.partial(_body, bn=bn, s=s)
    return pl.pallas_call(
        body,
        grid=(n // bn,),
        in_specs=[
            pl.BlockSpec((bn, d), lambda i: (i, 0)),
            pl.BlockSpec((bn * s, d), lambda i: (i, 0)),
            pl.BlockSpec((d, units), lambda i: (0, 0)),
            pl.BlockSpec((1, units), lambda i: (0, 0)),
        ],
        out_specs=pl.BlockSpec((bn, units), lambda i: (i, 0)),
        out_shape=jax.ShapeDtypeStruct((n, units), jnp.float32),
    )(x, nb_flat, kernel, bias2)
